# baseline (speedup 1.0000x reference)
import jax
import jax.numpy as jnp
from jax import lax
from jax.experimental import pallas as pl
from jax.experimental.pallas import tpu as pltpu

R = 64
N = 32768
LANES = 128
A = 4
NCH = 16
WIN = 4
CHC = N // NCH
CSL = CHC // LANES
SLICES = N // LANES


def _argmax_body(x_hbm, o_ref, xv, sems):
    def start(c):
        pltpu.make_async_copy(
            x_hbm.at[:, pl.ds(c * CHC, CHC)],
            xv.at[:, pl.ds(c * CHC, CHC)],
            sems.at[c % WIN],
        ).start()

    def wait(c):
        pltpu.make_async_copy(
            x_hbm.at[:, pl.ds(c * CHC, CHC)],
            xv.at[:, pl.ds(c * CHC, CHC)],
            sems.at[c % WIN],
        ).wait()

    for c in range(WIN):
        start(c)

    m = [jnp.full((R, LANES), -jnp.inf, jnp.float32) for _ in range(A)]
    i = [jnp.zeros((R, LANES), jnp.int32) for _ in range(A)]
    for c in range(NCH):
        wait(c)
        if c + WIN < NCH:
            start(c + WIN)
        for s in range(CSL):
            g = c * CSL + s
            a = g % A
            sl = xv[:, g * LANES:(g + 1) * LANES]
            gt = sl > m[a]
            m[a] = jnp.where(gt, sl, m[a])
            i[a] = jnp.where(gt, jnp.full((R, LANES), g, jnp.int32), i[a])

    lane = lax.broadcasted_iota(jnp.int32, (R, LANES), 1)
    mv = m[0]
    iv = i[0] * LANES + lane
    for a in range(1, A):
        ov = m[a]
        oi = i[a] * LANES + lane
        take = (ov > mv) | ((ov == mv) & (oi < iv))
        mv = jnp.where(take, ov, mv)
        iv = jnp.where(take, oi, iv)
    gmax = jnp.max(mv, axis=1, keepdims=True)
    cand = jnp.where(mv == gmax, iv, jnp.full((R, LANES), N, jnp.int32))
    o_ref[...] = jnp.min(cand, axis=1, keepdims=True)


@jax.jit
def kernel(X):
    out = pl.pallas_call(
        _argmax_body,
        in_specs=[pl.BlockSpec(memory_space=pl.ANY)],
        out_specs=pl.BlockSpec((R, 1), lambda: (0, 0)),
        out_shape=jax.ShapeDtypeStruct((R, 1), jnp.int32),
        scratch_shapes=[
            pltpu.VMEM((R, N), jnp.float32),
            pltpu.SemaphoreType.DMA((WIN,)),
        ],
    )(X)
    return out.reshape(R)


# speedup vs baseline: 1.2140x; 1.2140x over previous
import jax
import jax.numpy as jnp
from jax import lax
from jax.experimental import pallas as pl
from jax.experimental.pallas import tpu as pltpu

R = 64
N = 32768
LANES = 128
A = 8
RB = 8
NCH = R // RB
WIN = 4
SLICES = N // LANES


def _argmax_body(x_hbm, o_ref, xv, sems):
    def start(c):
        pltpu.make_async_copy(
            x_hbm.at[pl.ds(c * RB, RB)],
            xv.at[pl.ds(c * RB, RB)],
            sems.at[c % WIN],
        ).start()

    def wait(c):
        pltpu.make_async_copy(
            x_hbm.at[pl.ds(c * RB, RB)],
            xv.at[pl.ds(c * RB, RB)],
            sems.at[c % WIN],
        ).wait()

    for c in range(WIN):
        start(c)

    lane = lax.broadcasted_iota(jnp.int32, (RB, LANES), 1)
    for c in range(NCH):
        wait(c)
        if c + WIN < NCH:
            start(c + WIN)
        m = [jnp.full((RB, LANES), -jnp.inf, jnp.float32) for _ in range(A)]
        i = [jnp.zeros((RB, LANES), jnp.int32) for _ in range(A)]
        for s in range(SLICES):
            a = s % A
            sl = xv[c * RB:(c + 1) * RB, s * LANES:(s + 1) * LANES]
            gt = sl > m[a]
            m[a] = jnp.where(gt, sl, m[a])
            i[a] = jnp.where(gt, jnp.full((RB, LANES), s, jnp.int32), i[a])
        mv = m[0]
        iv = i[0] * LANES + lane
        for a in range(1, A):
            ov = m[a]
            oi = i[a] * LANES + lane
            take = (ov > mv) | ((ov == mv) & (oi < iv))
            mv = jnp.where(take, ov, mv)
            iv = jnp.where(take, oi, iv)
        gmax = jnp.max(mv, axis=1, keepdims=True)
        cand = jnp.where(mv == gmax, iv, jnp.full((RB, LANES), N, jnp.int32))
        o_ref[c * RB:(c + 1) * RB, :] = jnp.min(cand, axis=1, keepdims=True)


@jax.jit
def kernel(X):
    out = pl.pallas_call(
        _argmax_body,
        in_specs=[pl.BlockSpec(memory_space=pl.ANY)],
        out_specs=pl.BlockSpec((R, 1), lambda: (0, 0)),
        out_shape=jax.ShapeDtypeStruct((R, 1), jnp.int32),
        scratch_shapes=[
            pltpu.VMEM((R, N), jnp.float32),
            pltpu.SemaphoreType.DMA((WIN,)),
        ],
    )(X)
    return out.reshape(R)
